# baseline (speedup 1.0000x reference)
import jax
import jax.numpy as jnp
from jax.experimental import pallas as pl
from jax.experimental.pallas import tpu as pltpu

H = 16
N = 257
NREL = 964
B = 32
RB = 8
NIB = (N + RB - 1) // RB


def _fused(idx_ref, tab_ref, attn_ref, out_ref, bias_ref):
    @pl.when(pl.program_id(1) == 0)
    def _compute_bias():
        k = jax.lax.broadcasted_iota(jnp.int32, (NREL, N), 0)
        for r in range(RB):
            onehot = (k == idx_ref[r:r + 1, :]).astype(jnp.float32)
            bias_ref[:, r, :] = jnp.dot(tab_ref[...], onehot,
                                        preferred_element_type=jnp.float32,
                                        precision=jax.lax.Precision.HIGHEST)

    out_ref[...] = attn_ref[...] + bias_ref[...][None]


def kernel(attn, rel_pos_bias_table, rel_pos_index):
    tab_t = rel_pos_bias_table.T
    return pl.pallas_call(
        _fused,
        grid=(NIB, B),
        in_specs=[
            pl.BlockSpec((RB, N), lambda i, b: (i, 0)),
            pl.BlockSpec((H, NREL), lambda i, b: (0, 0)),
            pl.BlockSpec((1, H, RB, N), lambda i, b: (b, 0, i, 0)),
        ],
        out_specs=pl.BlockSpec((1, H, RB, N), lambda i, b: (b, 0, i, 0)),
        out_shape=jax.ShapeDtypeStruct(attn.shape, attn.dtype),
        scratch_shapes=[pltpu.VMEM((H, RB, N), jnp.float32)],
    )(rel_pos_index, tab_t, attn)


# speedup vs baseline: 1.3792x; 1.3792x over previous
import functools

import jax
import jax.numpy as jnp
from jax import lax
from jax.experimental import pallas as pl
from jax.experimental.pallas import tpu as pltpu
from jax.experimental.pallas import tpu_sc as plsc

H = 16
N = 257
NN = N * N
NREL = 964
B = 32
NW = 32
CPW = 2176
NNPAD = NW * CPW
TABW = NREL * H

CB = 8704
NCB = NNPAD // CB


def _sc_gather_body(tab_hbm, idx_hbm, out_hbm, tab_v, idx_v, outb_v):
    wid = lax.axis_index("s") * 2 + lax.axis_index("c")
    base = wid * CPW
    pltpu.sync_copy(tab_hbm, tab_v)
    pltpu.sync_copy(idx_hbm.at[pl.ds(base, CPW)], idx_v)

    def body(v, carry):
        col = v * 16
        addr = idx_v[pl.ds(col, 16)] * H
        for h in range(H):
            outb_v[h, pl.ds(col, 16)] = plsc.load_gather(tab_v, [addr + h])
        return carry

    lax.fori_loop(0, CPW // 16, body, 0)
    pltpu.sync_copy(outb_v, out_hbm.at[:, pl.ds(base, CPW)])


_sc_gather = functools.partial(
    pl.kernel,
    out_type=jax.ShapeDtypeStruct((H, NNPAD), jnp.float32),
    mesh=plsc.VectorSubcoreMesh(core_axis_name="c", subcore_axis_name="s",
                                num_cores=2, num_subcores=16),
    compiler_params=pltpu.CompilerParams(needs_layout_passes=False),
    scratch_types=[
        pltpu.VMEM((TABW,), jnp.float32),
        pltpu.VMEM((CPW,), jnp.int32),
        pltpu.VMEM((H, CPW), jnp.float32),
    ],
)(_sc_gather_body)


def _add_body(bias_ref, attn_ref, out_ref):
    out_ref[...] = attn_ref[...] + bias_ref[...][None]


def kernel(attn, rel_pos_bias_table, rel_pos_index):
    tab_flat = rel_pos_bias_table.reshape(-1)
    idx_flat = jnp.pad(rel_pos_index.reshape(-1), (0, NNPAD - NN))
    bias_t = _sc_gather(tab_flat, idx_flat)

    attn3 = attn.reshape(B, H, NN)
    out3 = pl.pallas_call(
        _add_body,
        grid=(NCB, B),
        in_specs=[
            pl.BlockSpec((H, CB), lambda c, b: (0, c)),
            pl.BlockSpec((1, H, CB), lambda c, b: (b, 0, c)),
        ],
        out_specs=pl.BlockSpec((1, H, CB), lambda c, b: (b, 0, c)),
        out_shape=jax.ShapeDtypeStruct((B, H, NN), attn.dtype),
    )(bias_t, attn3)
    return out3.reshape(B, H, N, N)


# speedup vs baseline: 2.3504x; 1.7042x over previous
import functools

import jax
import jax.numpy as jnp
from jax import lax
from jax.experimental import pallas as pl
from jax.experimental.pallas import tpu as pltpu
from jax.experimental.pallas import tpu_sc as plsc

H = 16
N = 257
NREL = 964
B = 32
TABW = NREL * H
RPW = 8
NV = 16

RB = 64
NIB = (N + RB - 1) // RB


def _gather_rows(tab_v, idx_v, outb_v, r):
    def body(c, carry):
        cs = c * 16
        addr = idx_v[r, pl.ds(cs, 16)] * H
        for h in range(H):
            outb_v[h, r, pl.ds(cs, 16)] = plsc.load_gather(tab_v, [addr + h])
        return carry
    lax.fori_loop(0, NV, body, 0)
    lanes = lax.iota(jnp.int32, 16)
    rv = jnp.full((16,), r, jnp.int32)
    cv = jnp.full((16,), N - 1, jnp.int32)
    iv = plsc.load_gather(idx_v, [rv, cv])
    vals = plsc.load_gather(tab_v, [iv * H + lanes])
    plsc.store_scatter(outb_v, [lanes, rv, cv], vals)


def _sc_gather_body(tab_hbm, idx_hbm, out_hbm, tab_v, idx_v, outb_v):
    wid = lax.axis_index("s") * 2 + lax.axis_index("c")
    row0 = wid * RPW
    pltpu.sync_copy(tab_hbm, tab_v)
    pltpu.sync_copy(idx_hbm.at[pl.ds(row0, RPW), :], idx_v.at[:RPW])

    def body(r, carry):
        _gather_rows(tab_v, idx_v, outb_v, r)
        return carry
    lax.fori_loop(0, RPW, body, 0)
    pltpu.sync_copy(outb_v.at[:, :RPW], out_hbm.at[:, pl.ds(row0, RPW), :])

    @pl.when(wid == NW - 1)
    def _tail_row():
        pltpu.sync_copy(idx_hbm.at[pl.ds(N - 1, 1), :], idx_v.at[RPW:])
        _gather_rows(tab_v, idx_v, outb_v, RPW)
        pltpu.sync_copy(outb_v.at[:, RPW:], out_hbm.at[:, pl.ds(N - 1, 1), :])


NW = 32

_sc_gather = functools.partial(
    pl.kernel,
    out_type=jax.ShapeDtypeStruct((H, N, N), jnp.float32),
    mesh=plsc.VectorSubcoreMesh(core_axis_name="c", subcore_axis_name="s",
                                num_cores=2, num_subcores=16),
    compiler_params=pltpu.CompilerParams(needs_layout_passes=False),
    scratch_types=[
        pltpu.VMEM((TABW,), jnp.float32),
        pltpu.VMEM((RPW + 1, N), jnp.int32),
        pltpu.VMEM((H, RPW + 1, N), jnp.float32),
    ],
)(_sc_gather_body)


def _add_body(bias_ref, attn_ref, out_ref):
    out_ref[...] = attn_ref[...] + bias_ref[...][None]


def kernel(attn, rel_pos_bias_table, rel_pos_index):
    tab_flat = rel_pos_bias_table.reshape(-1)
    bias3 = _sc_gather(tab_flat, rel_pos_index)

    return pl.pallas_call(
        _add_body,
        grid=(NIB, B),
        in_specs=[
            pl.BlockSpec((H, RB, N), lambda i, b: (0, i, 0)),
            pl.BlockSpec((1, H, RB, N), lambda i, b: (b, 0, i, 0)),
        ],
        out_specs=pl.BlockSpec((1, H, RB, N), lambda i, b: (b, 0, i, 0)),
        out_shape=jax.ShapeDtypeStruct(attn.shape, attn.dtype),
    )(bias3, attn)


# speedup vs baseline: 2.4724x; 1.0519x over previous
import functools

import jax
import jax.numpy as jnp
from jax import lax
from jax.experimental import pallas as pl
from jax.experimental.pallas import tpu as pltpu
from jax.experimental.pallas import tpu_sc as plsc

H = 16
N = 257
NREL = 964
B = 32
TABW = NREL * H
RPW = 8
NV = 16

RB = 128
NIB = (N + RB - 1) // RB


def _gather_rows(tab_v, idx_v, outb_v, r):
    def body(c, carry):
        cs = c * 16
        addr = idx_v[r, pl.ds(cs, 16)] * H
        for h in range(H):
            outb_v[h, r, pl.ds(cs, 16)] = plsc.load_gather(tab_v, [addr + h])
        return carry
    lax.fori_loop(0, NV, body, 0)
    lanes = lax.iota(jnp.int32, 16)
    rv = jnp.full((16,), r, jnp.int32)
    cv = jnp.full((16,), N - 1, jnp.int32)
    iv = plsc.load_gather(idx_v, [rv, cv])
    vals = plsc.load_gather(tab_v, [iv * H + lanes])
    plsc.store_scatter(outb_v, [lanes, rv, cv], vals)


def _sc_gather_body(tab_hbm, idx_hbm, out_hbm, tab_v, idx_v, outb_v):
    wid = lax.axis_index("s") * 2 + lax.axis_index("c")
    row0 = wid * RPW
    pltpu.sync_copy(tab_hbm, tab_v)
    pltpu.sync_copy(idx_hbm.at[pl.ds(row0, RPW), :], idx_v.at[:RPW])

    def body(r, carry):
        _gather_rows(tab_v, idx_v, outb_v, r)
        return carry
    lax.fori_loop(0, RPW, body, 0)
    pltpu.sync_copy(outb_v.at[:, :RPW], out_hbm.at[:, pl.ds(row0, RPW), :])

    @pl.when(wid == NW - 1)
    def _tail_row():
        pltpu.sync_copy(idx_hbm.at[pl.ds(N - 1, 1), :], idx_v.at[RPW:])
        _gather_rows(tab_v, idx_v, outb_v, RPW)
        pltpu.sync_copy(outb_v.at[:, RPW:], out_hbm.at[:, pl.ds(N - 1, 1), :])


NW = 32

_sc_gather = functools.partial(
    pl.kernel,
    out_type=jax.ShapeDtypeStruct((H, N, N), jnp.float32),
    mesh=plsc.VectorSubcoreMesh(core_axis_name="c", subcore_axis_name="s",
                                num_cores=2, num_subcores=16),
    compiler_params=pltpu.CompilerParams(needs_layout_passes=False),
    scratch_types=[
        pltpu.VMEM((TABW,), jnp.float32),
        pltpu.VMEM((RPW + 1, N), jnp.int32),
        pltpu.VMEM((H, RPW + 1, N), jnp.float32),
    ],
)(_sc_gather_body)


def _add_body(bias_ref, attn_ref, out_ref):
    out_ref[...] = attn_ref[...] + bias_ref[...][None]


def kernel(attn, rel_pos_bias_table, rel_pos_index):
    tab_flat = rel_pos_bias_table.reshape(-1)
    bias3 = _sc_gather(tab_flat, rel_pos_index)

    return pl.pallas_call(
        _add_body,
        grid=(NIB, B),
        in_specs=[
            pl.BlockSpec((H, RB, N), lambda i, b: (0, i, 0)),
            pl.BlockSpec((1, H, RB, N), lambda i, b: (b, 0, i, 0)),
        ],
        out_specs=pl.BlockSpec((1, H, RB, N), lambda i, b: (b, 0, i, 0)),
        out_shape=jax.ShapeDtypeStruct(attn.shape, attn.dtype),
    )(bias3, attn)


# speedup vs baseline: 2.6054x; 1.0538x over previous
import functools

import jax
import jax.numpy as jnp
from jax import lax
from jax.experimental import pallas as pl
from jax.experimental.pallas import tpu as pltpu
from jax.experimental.pallas import tpu_sc as plsc

H = 16
N = 257
NREL = 964
B = 32
TABW = NREL * H
RPW = 8
NV = 16

RB = 8
NIB = (N + RB - 1) // RB


def _gather_rows(tab_v, idx_v, outb_v, r):
    def body(c, carry):
        cs = c * 16
        addr = idx_v[r, pl.ds(cs, 16)] * H
        for h in range(H):
            outb_v[h, r, pl.ds(cs, 16)] = plsc.load_gather(tab_v, [addr + h])
        return carry
    lax.fori_loop(0, NV, body, 0)
    lanes = lax.iota(jnp.int32, 16)
    rv = jnp.full((16,), r, jnp.int32)
    cv = jnp.full((16,), N - 1, jnp.int32)
    iv = plsc.load_gather(idx_v, [rv, cv])
    vals = plsc.load_gather(tab_v, [iv * H + lanes])
    plsc.store_scatter(outb_v, [lanes, rv, cv], vals)


def _sc_gather_body(tab_hbm, idx_hbm, out_hbm, tab_v, idx_v, outb_v):
    wid = lax.axis_index("s") * 2 + lax.axis_index("c")
    row0 = wid * RPW
    pltpu.sync_copy(tab_hbm, tab_v)
    pltpu.sync_copy(idx_hbm.at[pl.ds(row0, RPW), :], idx_v.at[:RPW])

    def body(r, carry):
        _gather_rows(tab_v, idx_v, outb_v, r)
        return carry
    lax.fori_loop(0, RPW, body, 0)
    pltpu.sync_copy(outb_v.at[:, :RPW], out_hbm.at[:, pl.ds(row0, RPW), :])

    @pl.when(wid == NW - 1)
    def _tail_row():
        pltpu.sync_copy(idx_hbm.at[pl.ds(N - 1, 1), :], idx_v.at[RPW:])
        _gather_rows(tab_v, idx_v, outb_v, RPW)
        pltpu.sync_copy(outb_v.at[:, RPW:], out_hbm.at[:, pl.ds(N - 1, 1), :])


NW = 32

_sc_gather = functools.partial(
    pl.kernel,
    out_type=jax.ShapeDtypeStruct((H, N, N), jnp.float32),
    mesh=plsc.VectorSubcoreMesh(core_axis_name="c", subcore_axis_name="s",
                                num_cores=2, num_subcores=16),
    compiler_params=pltpu.CompilerParams(needs_layout_passes=False),
    scratch_types=[
        pltpu.VMEM((TABW,), jnp.float32),
        pltpu.VMEM((RPW + 1, N), jnp.int32),
        pltpu.VMEM((H, RPW + 1, N), jnp.float32),
    ],
)(_sc_gather_body)


def _add_body(bias_ref, attn_ref, out_ref):
    out_ref[...] = attn_ref[...] + bias_ref[...][None]


def kernel(attn, rel_pos_bias_table, rel_pos_index):
    tab_flat = rel_pos_bias_table.reshape(-1)
    bias3 = _sc_gather(tab_flat, rel_pos_index)

    return pl.pallas_call(
        _add_body,
        grid=(NIB,),
        in_specs=[
            pl.BlockSpec((H, RB, N), lambda i: (0, i, 0)),
            pl.BlockSpec((B, H, RB, N), lambda i: (0, 0, i, 0)),
        ],
        out_specs=pl.BlockSpec((B, H, RB, N), lambda i: (0, 0, i, 0)),
        out_shape=jax.ShapeDtypeStruct(attn.shape, attn.dtype),
        compiler_params=pltpu.CompilerParams(vmem_limit_bytes=100*1024*1024),
    )(bias3, attn)


# speedup vs baseline: 2.6075x; 1.0008x over previous
import functools

import jax
import jax.numpy as jnp
from jax import lax
from jax.experimental import pallas as pl
from jax.experimental.pallas import tpu as pltpu
from jax.experimental.pallas import tpu_sc as plsc

H = 16
N = 257
NREL = 964
B = 32
TABW = NREL * H
RPW = 8
NV = 16

RB = 8
NIB = (N + RB - 1) // RB


def _gather_rows(tab_v, idx_v, outb_v, r):
    def body(c, carry):
        cs = c * 16
        addr = idx_v[r, pl.ds(cs, 16)] * H
        for h in range(H):
            outb_v[h, r, pl.ds(cs, 16)] = plsc.load_gather(tab_v, [addr + h])
        return carry
    lax.fori_loop(0, NV, body, 0)
    lanes = lax.iota(jnp.int32, 16)
    rv = jnp.full((16,), r, jnp.int32)
    cv = jnp.full((16,), N - 1, jnp.int32)
    iv = plsc.load_gather(idx_v, [rv, cv])
    vals = plsc.load_gather(tab_v, [iv * H + lanes])
    plsc.store_scatter(outb_v, [lanes, rv, cv], vals)


def _sc_gather_body(tab_hbm, idx_hbm, out_hbm, tab_v, idx_v, outb_v):
    wid = lax.axis_index("s") * 2 + lax.axis_index("c")
    row0 = wid * RPW
    pltpu.sync_copy(tab_hbm, tab_v)
    pltpu.sync_copy(idx_hbm.at[pl.ds(row0, RPW), :], idx_v.at[:RPW])

    def body(r, carry):
        _gather_rows(tab_v, idx_v, outb_v, r)
        return carry
    lax.fori_loop(0, RPW, body, 0)
    pltpu.sync_copy(outb_v.at[:, :RPW], out_hbm.at[:, pl.ds(row0, RPW), :])

    @pl.when(wid == NW - 1)
    def _tail_row():
        pltpu.sync_copy(idx_hbm.at[pl.ds(N - 1, 1), :], idx_v.at[RPW:])
        _gather_rows(tab_v, idx_v, outb_v, RPW)
        pltpu.sync_copy(outb_v.at[:, RPW:], out_hbm.at[:, pl.ds(N - 1, 1), :])


NW = 32

_sc_gather = functools.partial(
    pl.kernel,
    out_type=jax.ShapeDtypeStruct((H, N, N), jnp.float32),
    mesh=plsc.VectorSubcoreMesh(core_axis_name="c", subcore_axis_name="s",
                                num_cores=2, num_subcores=16),
    compiler_params=pltpu.CompilerParams(needs_layout_passes=False),
    scratch_types=[
        pltpu.VMEM((TABW,), jnp.float32),
        pltpu.VMEM((RPW + 1, N), jnp.int32),
        pltpu.VMEM((H, RPW + 1, N), jnp.float32),
    ],
)(_sc_gather_body)


def _add_manual(bias_hbm, attn_hbm, out_hbm,
                bias_v, a0, a1, o0, o1, bsem, ia0, ia1, oa0, oa1):
    b = pl.program_id(0)

    @pl.when(b == 0)
    def _prologue():
        pltpu.async_copy(bias_hbm, bias_v, bsem).wait()
        pltpu.async_copy(attn_hbm.at[0], a0, ia0)

    def _step(abuf, obuf, isem, osem, nbuf, nsem):
        @pl.when(b + 1 < B)
        def _prefetch():
            pltpu.async_copy(attn_hbm.at[b + 1], nbuf, nsem)

        pltpu.make_async_copy(attn_hbm.at[b], abuf, isem).wait()

        @pl.when(b >= 2)
        def _wait_prev_out():
            pltpu.make_async_copy(obuf, out_hbm.at[b - 2], osem).wait()

        obuf[...] = abuf[...] + bias_v[...]
        pltpu.async_copy(obuf, out_hbm.at[b], osem)

    @pl.when(b % 2 == 0)
    def _even():
        _step(a0, o0, ia0, oa0, a1, ia1)

    @pl.when(b % 2 == 1)
    def _odd():
        _step(a1, o1, ia1, oa1, a0, ia0)

    @pl.when(b == B - 1)
    def _drain():
        pltpu.make_async_copy(o0, out_hbm.at[B - 2], oa0).wait()
        pltpu.make_async_copy(o1, out_hbm.at[B - 1], oa1).wait()


def kernel(attn, rel_pos_bias_table, rel_pos_index):
    tab_flat = rel_pos_bias_table.reshape(-1)
    bias3 = _sc_gather(tab_flat, rel_pos_index)

    vmem = lambda: pltpu.VMEM((H, N, N), jnp.float32)
    return pl.pallas_call(
        _add_manual,
        grid=(B,),
        in_specs=[
            pl.BlockSpec(memory_space=pltpu.HBM),
            pl.BlockSpec(memory_space=pltpu.HBM),
        ],
        out_specs=pl.BlockSpec(memory_space=pltpu.HBM),
        out_shape=jax.ShapeDtypeStruct(attn.shape, attn.dtype),
        scratch_shapes=[vmem(), vmem(), vmem(), vmem(), vmem(),
                        pltpu.SemaphoreType.DMA, pltpu.SemaphoreType.DMA,
                        pltpu.SemaphoreType.DMA, pltpu.SemaphoreType.DMA,
                        pltpu.SemaphoreType.DMA],
        compiler_params=pltpu.CompilerParams(vmem_limit_bytes=100 * 1024 * 1024),
    )(bias3, attn)
